# baseline (speedup 1.0000x reference)
import functools

import jax
import jax.numpy as jnp
from jax import lax
from jax.experimental import pallas as pl
from jax.experimental.pallas import tpu as pltpu
from jax.experimental.pallas import tpu_sc as plsc

B, S, L, MM, H = 8, 3, 2048, 1024, 1024
NC, NS = 2, 16
NW = NC * NS
NSEG = B * S
VL = 16
COL_U = 4

RSC = 896
NQ = 4
NTASK = NSEG * NQ
TPW = NTASK // NW
RPT = RSC // NQ
CH = 32
NCH = RPT // CH

LTC = L - RSC
TCB = 576
NLB = LTC // TCB
NSTR = 2


def _sc_partial_sums(x2, mf_sc):
    mesh = plsc.VectorSubcoreMesh(
        core_axis_name="c", subcore_axis_name="s", num_cores=NC, num_subcores=NS
    )

    @functools.partial(
        pl.kernel,
        out_type=jax.ShapeDtypeStruct((NQ, NSEG, MM), jnp.float32),
        mesh=mesh,
        scratch_types=[
            pltpu.VMEM((CH, MM), jnp.float32),
            pltpu.VMEM((CH, MM), jnp.float32),
            pltpu.VMEM((CH, MM), jnp.float32),
            pltpu.VMEM((RPT,), jnp.float32),
            pltpu.VMEM((MM,), jnp.float32),
            pltpu.SemaphoreType.DMA,
            pltpu.SemaphoreType.DMA,
            pltpu.SemaphoreType.DMA,
        ],
    )
    def k(x_hbm, mf_hbm, out_hbm, buf0, buf1, buf2, mfb, acc,
          sem0, sem1, sem2):
        wid = lax.axis_index("s") * NC + lax.axis_index("c")

        def start(ch_row0, buf, sem):
            pltpu.make_async_copy(
                x_hbm.at[pl.ds(ch_row0, CH)], buf, sem
            ).start()

        def wait(buf, sem):
            pltpu.make_async_copy(x_hbm.at[pl.ds(0, CH)], buf, sem).wait()

        def compute(buf, moff):
            def row_body(rg, _):
                r0 = rg * VL
                mvec = mfb[pl.ds(moff + r0, VL)]
                ms = [mvec[j] for j in range(VL)]

                def col_body(cb, _):
                    for cs in range(COL_U):
                        o = cb * (COL_U * VL) + cs * VL
                        v = acc[pl.ds(o, VL)]
                        for j in range(VL):
                            v = v + buf[r0 + j, pl.ds(o, VL)] * ms[j]
                        acc[pl.ds(o, VL)] = v
                    return 0

                lax.fori_loop(0, MM // (COL_U * VL), col_body, 0)
                return 0

            lax.fori_loop(0, CH // VL, row_body, 0)

        def task_body(ti, _):
            t = wid * TPW + ti
            seg = t // NQ
            q = t - seg * NQ
            row0 = seg * L + LTC + q * RPT
            pltpu.sync_copy(mf_hbm.at[t], mfb)

            def zero_body(c, _):
                acc[pl.ds(c * VL, VL)] = jnp.zeros((VL,), jnp.float32)
                return 0

            lax.fori_loop(0, MM // VL, zero_body, 0)

            rings = ((buf0, sem0), (buf1, sem1), (buf2, sem2))
            start(row0, buf0, sem0)
            start(row0 + CH, buf1, sem1)

            def chunk_body(c, _):
                for r in range(3):
                    @pl.when(c % 3 == r)
                    def _():
                        buf, sem = rings[r]
                        buf_n, sem_n = rings[(r + 2) % 3]
                        wait(buf, sem)
                        compute(buf, c * CH)

                        @pl.when(c + 2 < NCH)
                        def _():
                            start(row0 + (c + 2) * CH, buf_n, sem_n)

                return 0

            lax.fori_loop(0, NCH, chunk_body, 0)
            pltpu.sync_copy(acc, out_hbm.at[q, seg])
            return 0

        lax.fori_loop(0, TPW, task_body, 0)

    return k(x2, mf_sc)


def _tc_partial_sums(x24, mf_row):
    sub = TCB // NSTR

    def body(mf_ref, *rest):
        x_refs, out_ref = rest[:NSTR], rest[NSTR]
        lb = pl.program_id(1)

        @pl.when(lb == 0)
        def _():
            out_ref[...] = jnp.zeros_like(out_ref)

        mrow = mf_ref[0]
        mwin = mrow[:, 0:TCB]
        for w in range(1, NLB):
            mwin = jnp.where(lb == w, mrow[:, w * TCB:(w + 1) * TCB], mwin)
        part = out_ref[...]
        for k in range(NSTR):
            mk = mwin[:, k * sub:(k + 1) * sub]
            part += lax.dot_general(
                mk, x_refs[k][0], (((1,), (0,)), ((), ())),
                precision=lax.Precision.HIGHEST,
            )[None]
        out_ref[...] = part

    def x_spec(k):
        return pl.BlockSpec(
            (1, sub, MM),
            lambda s_i, l_i: (s_i, NSTR * l_i + k, 0),
        )

    return pl.pallas_call(
        body,
        grid=(NSEG, NLB),
        in_specs=[
            pl.BlockSpec((1, 1, L), lambda s_i, l_i: (s_i, 0, 0)),
        ] + [x_spec(k) for k in range(NSTR)],
        out_specs=pl.BlockSpec((1, 1, MM), lambda s_i, l_i: (s_i, 0, 0)),
        out_shape=jax.ShapeDtypeStruct((NSEG, 1, MM), jnp.float32),
    )(mf_row, *([x24] * NSTR))


def _tc_finish(part_sc, part_tc, mf24, gf_col, w_proj, b_proj, wc24, bc24):
    def body(psc_ref, ptc_ref, mf_ref, gf_ref, wp_ref, bp_ref, wc_ref, bc_ref,
             out_ref):
        seg_sum = (psc_ref[0] + psc_ref[1] + psc_ref[2] + psc_ref[3]
                   + ptc_ref[:, 0])
        den = jnp.sum(mf_ref[:], axis=1, keepdims=True)
        logits = lax.dot_general(
            wc_ref[:], gf_ref[:], (((1,), (0,)), ((), ())),
            precision=lax.Precision.HIGHEST,
        ) + bc_ref[:]
        m = jnp.max(logits, axis=0, keepdims=True)
        e = jnp.exp(logits - m)
        s = jnp.sum(e, axis=0, keepdims=True) / B
        w24 = e / s
        scaled = seg_sum * (w24 / den)
        ri = lax.broadcasted_iota(jnp.int32, (B, NSEG), 0)
        cj = lax.broadcasted_iota(jnp.int32, (B, NSEG), 1)
        sel = jnp.where((cj >= S * ri) & (cj < S * ri + S), 1.0, 0.0)
        xw = lax.dot_general(
            sel, scaled, (((1,), (0,)), ((), ())),
            precision=lax.Precision.HIGHEST,
        )
        out_ref[:] = lax.dot_general(
            xw, wp_ref[:], (((1,), (1,)), ((), ())),
            precision=lax.Precision.HIGHEST,
        ) + bp_ref[:]

    return pl.pallas_call(
        body, out_shape=jax.ShapeDtypeStruct((B, H), jnp.float32)
    )(part_sc, part_tc, mf24, gf_col, w_proj, b_proj, wc24, bc24)


@jax.jit
def kernel(graph_feature, x_tensors, x_mask, W_proj, b_proj, W_comb, b_comb):
    mf = (~x_mask).astype(jnp.float32)
    x2 = x_tensors.reshape(B * S * L, MM)
    x24 = x_tensors.reshape(NSEG, L, MM)
    mf24 = mf.reshape(NSEG, L)
    mf_sc = mf24[:, LTC:].reshape(NTASK, RPT)
    part_sc = _sc_partial_sums(x2, mf_sc)
    mf_row = mf.reshape(NSEG, 1, L)
    part_tc = _tc_partial_sums(x24, mf_row)
    gf_col = graph_feature.reshape(MM, 1)
    wc24 = jnp.tile(W_comb, (B, 1))
    bc24 = jnp.tile(b_comb, (B,)).reshape(NSEG, 1)
    bp = b_proj.reshape(1, H)
    return _tc_finish(part_sc, part_tc, mf24, gf_col, W_proj, bp, wc24, bc24)


# speedup vs baseline: 1.0045x; 1.0045x over previous
import functools

import jax
import jax.numpy as jnp
from jax import lax
from jax.experimental import pallas as pl
from jax.experimental.pallas import tpu as pltpu
from jax.experimental.pallas import tpu_sc as plsc

B, S, L, MM, H = 8, 3, 2048, 1024, 1024
NC, NS = 2, 16
NW = NC * NS
NSEG = B * S
VL = 16
COL_U = 4

RSC = 896
NQ = 4
NTASK = NSEG * NQ
TPW = NTASK // NW
RPT = RSC // NQ
CH = 32
NCH = RPT // CH

LTC = L - RSC
TCB = 576
NLB = LTC // TCB
NSTR = 2


def _sc_partial_sums(x2, mf_sc):
    mesh = plsc.VectorSubcoreMesh(
        core_axis_name="c", subcore_axis_name="s", num_cores=NC, num_subcores=NS
    )

    @functools.partial(
        pl.kernel,
        out_type=jax.ShapeDtypeStruct((NQ, NSEG, MM), jnp.float32),
        mesh=mesh,
        scratch_types=[
            pltpu.VMEM((CH, MM), jnp.float32),
            pltpu.VMEM((CH, MM), jnp.float32),
            pltpu.VMEM((RPT,), jnp.float32),
            pltpu.VMEM((MM,), jnp.float32),
            pltpu.SemaphoreType.DMA,
            pltpu.SemaphoreType.DMA,
        ],
    )
    def k(x_hbm, mf_hbm, out_hbm, buf0, buf1, mfb, acc, sem0, sem1):
        wid = lax.axis_index("s") * NC + lax.axis_index("c")

        def start(ch_row0, buf, sem):
            pltpu.make_async_copy(
                x_hbm.at[pl.ds(ch_row0, CH)], buf, sem
            ).start()

        def wait(buf, sem):
            pltpu.make_async_copy(x_hbm.at[pl.ds(0, CH)], buf, sem).wait()

        def compute(buf, moff):
            def row_body(rg, _):
                r0 = rg * VL
                mvec = mfb[pl.ds(moff + r0, VL)]
                ms = [mvec[j] for j in range(VL)]

                def col_body(cb, _):
                    for cs in range(COL_U):
                        o = cb * (COL_U * VL) + cs * VL
                        v = acc[pl.ds(o, VL)]
                        for j in range(VL):
                            v = v + buf[r0 + j, pl.ds(o, VL)] * ms[j]
                        acc[pl.ds(o, VL)] = v
                    return 0

                lax.fori_loop(0, MM // (COL_U * VL), col_body, 0)
                return 0

            lax.fori_loop(0, CH // VL, row_body, 0)

        def task_body(ti, _):
            t = wid * TPW + ti
            seg = t // NQ
            q = t - seg * NQ
            row0 = seg * L + LTC + q * RPT
            pltpu.sync_copy(mf_hbm.at[t], mfb)

            def zero_body(c, _):
                acc[pl.ds(c * VL, VL)] = jnp.zeros((VL,), jnp.float32)
                return 0

            lax.fori_loop(0, MM // VL, zero_body, 0)

            start(row0, buf0, sem0)

            def pair_body(cp, _):
                c0 = cp * 2
                start(row0 + (c0 + 1) * CH, buf1, sem1)
                wait(buf0, sem0)
                compute(buf0, c0 * CH)

                @pl.when(c0 + 2 < NCH)
                def _():
                    start(row0 + (c0 + 2) * CH, buf0, sem0)

                wait(buf1, sem1)
                compute(buf1, (c0 + 1) * CH)
                return 0

            lax.fori_loop(0, NCH // 2, pair_body, 0)
            if NCH % 2 == 1:
                wait(buf0, sem0)
                compute(buf0, (NCH - 1) * CH)
            pltpu.sync_copy(acc, out_hbm.at[q, seg])
            return 0

        lax.fori_loop(0, TPW, task_body, 0)

    return k(x2, mf_sc)


def _tc_partial_sums(x24, mf_row):
    sub = TCB // NSTR

    def body(mf_ref, *rest):
        x_refs, out_ref = rest[:NSTR], rest[NSTR]
        lb = pl.program_id(1)

        @pl.when(lb == 0)
        def _():
            out_ref[...] = jnp.zeros_like(out_ref)

        mrow = mf_ref[0]
        mwin = mrow[:, 0:TCB]
        for w in range(1, NLB):
            mwin = jnp.where(lb == w, mrow[:, w * TCB:(w + 1) * TCB], mwin)
        part = out_ref[...]
        for k in range(NSTR):
            mk = mwin[:, k * sub:(k + 1) * sub]
            part += lax.dot_general(
                mk, x_refs[k][0], (((1,), (0,)), ((), ())),
                precision=lax.Precision.HIGHEST,
            )[None]
        out_ref[...] = part

    def x_spec(k):
        return pl.BlockSpec(
            (1, sub, MM),
            lambda s_i, l_i: (s_i, NSTR * l_i + k, 0),
        )

    return pl.pallas_call(
        body,
        grid=(NSEG, NLB),
        in_specs=[
            pl.BlockSpec((1, 1, L), lambda s_i, l_i: (s_i, 0, 0)),
        ] + [x_spec(k) for k in range(NSTR)],
        out_specs=pl.BlockSpec((1, 1, MM), lambda s_i, l_i: (s_i, 0, 0)),
        out_shape=jax.ShapeDtypeStruct((NSEG, 1, MM), jnp.float32),
    )(mf_row, *([x24] * NSTR))


def _tc_finish(part_sc, part_tc, mf24, gf_col, w_proj, b_proj, wc24, bc24):
    def body(psc_ref, ptc_ref, mf_ref, gf_ref, wp_ref, bp_ref, wc_ref, bc_ref,
             out_ref):
        seg_sum = (psc_ref[0] + psc_ref[1] + psc_ref[2] + psc_ref[3]
                   + ptc_ref[:, 0])
        den = jnp.sum(mf_ref[:], axis=1, keepdims=True)
        logits = lax.dot_general(
            wc_ref[:], gf_ref[:], (((1,), (0,)), ((), ())),
            precision=lax.Precision.HIGHEST,
        ) + bc_ref[:]
        m = jnp.max(logits, axis=0, keepdims=True)
        e = jnp.exp(logits - m)
        s = jnp.sum(e, axis=0, keepdims=True) / B
        w24 = e / s
        scaled = seg_sum * (w24 / den)
        ri = lax.broadcasted_iota(jnp.int32, (B, NSEG), 0)
        cj = lax.broadcasted_iota(jnp.int32, (B, NSEG), 1)
        sel = jnp.where((cj >= S * ri) & (cj < S * ri + S), 1.0, 0.0)
        xw = lax.dot_general(
            sel, scaled, (((1,), (0,)), ((), ())),
            precision=lax.Precision.HIGHEST,
        )
        out_ref[:] = lax.dot_general(
            xw, wp_ref[:], (((1,), (1,)), ((), ())),
            precision=lax.Precision.HIGHEST,
        ) + bp_ref[:]

    return pl.pallas_call(
        body, out_shape=jax.ShapeDtypeStruct((B, H), jnp.float32)
    )(part_sc, part_tc, mf24, gf_col, w_proj, b_proj, wc24, bc24)


@jax.jit
def kernel(graph_feature, x_tensors, x_mask, W_proj, b_proj, W_comb, b_comb):
    mf = (~x_mask).astype(jnp.float32)
    x2 = x_tensors.reshape(B * S * L, MM)
    x24 = x_tensors.reshape(NSEG, L, MM)
    mf24 = mf.reshape(NSEG, L)
    mf_sc = mf24[:, LTC:].reshape(NTASK, RPT)
    part_sc = _sc_partial_sums(x2, mf_sc)
    mf_row = mf.reshape(NSEG, 1, L)
    part_tc = _tc_partial_sums(x24, mf_row)
    gf_col = graph_feature.reshape(MM, 1)
    wc24 = jnp.tile(W_comb, (B, 1))
    bc24 = jnp.tile(b_comb, (B,)).reshape(NSEG, 1)
    bp = b_proj.reshape(1, H)
    return _tc_finish(part_sc, part_tc, mf24, gf_col, W_proj, bp, wc24, bc24)
